# baseline (speedup 1.0000x reference)
import jax
import jax.numpy as jnp
from jax.experimental import pallas as pl
from jax.experimental.pallas import tpu as pltpu

_R = 32


def _tc_gather(table, idx3):
  g, one, r = idx3.shape
  d = table.shape[1]
  b_total = g * r

  def body(idx_ref, table_ref, o_ref, sem):
    def issue(i):
      row = idx_ref[0, 0, i]
      pltpu.make_async_copy(table_ref.at[pl.ds(row, 1)],
                            o_ref.at[pl.ds(i, 1)], sem).start()

    @pl.loop(0, r)
    def _(i):
      issue(i)

    @pl.loop(0, r)
    def _(i):
      pltpu.make_async_copy(table_ref.at[pl.ds(0, 1)],
                            o_ref.at[pl.ds(0, 1)], sem).wait()

  return pl.pallas_call(
      body,
      grid=(g,),
      in_specs=[
          pl.BlockSpec((1, 1, r), lambda i: (i, 0, 0),
                       memory_space=pltpu.SMEM),
          pl.BlockSpec(memory_space=pl.ANY),
      ],
      out_specs=pl.BlockSpec((r, d), lambda i: (i, 0)),
      out_shape=jax.ShapeDtypeStruct((b_total, d), jnp.float32),
      scratch_shapes=[pltpu.SemaphoreType.DMA],
  )(idx3, table)


def kernel(x, table):
  b, t = x.shape
  vocab = table.shape[0]
  idx = x.reshape(-1).astype(jnp.int32)
  b_total = idx.shape[0]
  idx3 = idx.reshape(b_total // _R, 1, _R)
  out = _tc_gather(table, idx3)
  return out.reshape(b, t, vocab)


# speedup vs baseline: 1.1460x; 1.1460x over previous
import functools

import jax
import jax.numpy as jnp
from jax import lax
from jax.experimental import pallas as pl
from jax.experimental.pallas import tpu as pltpu
import jax.experimental.pallas.tpu_sc as plsc

_NC = 2
_NS = 16
_NW = _NC * _NS

_CH = 4
_R = 32
_TC_ROWS = 2560


def _sc_gather(table, idx3):
  nw, nch, ch = idx3.shape
  d = table.shape[1]
  b_total = nw * nch * ch
  assert nch >= 2 and nch % 2 == 0
  mesh = plsc.VectorSubcoreMesh(core_axis_name="c", subcore_axis_name="s")

  @functools.partial(
      pl.kernel,
      out_type=jax.ShapeDtypeStruct((b_total, d), jnp.float32),
      mesh=mesh,
      scratch_types=[
          pltpu.VMEM((nch, ch), jnp.int32),
          pltpu.VMEM((ch, d), jnp.float32),
          pltpu.VMEM((ch, d), jnp.float32),
          pltpu.SemaphoreType.DMA,
          pltpu.SemaphoreType.DMA,
          pltpu.SemaphoreType.DMA,
          pltpu.SemaphoreType.DMA,
      ],
  )
  def k(table_hbm, idx_hbm, out_hbm, idx_v, buf0, buf1, gs0, gs1, ss0, ss1):
    bufs = (buf0, buf1)
    gsems = (gs0, gs1)
    ssems = (ss0, ss1)
    wid = lax.axis_index("s") * _NC + lax.axis_index("c")
    base_row = wid * (nch * ch)

    pltpu.sync_copy(idx_hbm.at[wid], idx_v)

    def gather_start(b, g):
      pltpu.async_copy(table_hbm.at[idx_v.at[g]], bufs[b], gsems[b])

    def gather_wait(b):
      pltpu.make_async_copy(table_hbm.at[idx_v.at[0]], bufs[b],
                            gsems[b]).wait()

    def scatter_start(b, g):
      pltpu.async_copy(bufs[b], out_hbm.at[pl.ds(base_row + g * ch, ch)],
                       ssems[b])

    def scatter_wait(b):
      pltpu.make_async_copy(bufs[b], out_hbm.at[pl.ds(0, ch)],
                            ssems[b]).wait()

    def step(g, b, first=False, last=False):
      gather_wait(b)
      scatter_start(b, g)
      if not first:
        scatter_wait(1 - b)
      if not last:
        gather_start(1 - b, g + 1)

    gather_start(0, 0)
    step(0, 0, first=True)

    @pl.loop(0, (nch - 2) // 2)
    def _(o):
      step(2 * o + 1, 1)
      step(2 * o + 2, 0)

    step(nch - 1, 1, last=True)
    scatter_wait(1)

  return k(table, idx3)


def _tc_gather(table, idx3):
  g, one, r = idx3.shape
  d = table.shape[1]
  b_total = g * r

  def body(idx_ref, table_ref, o_ref, sem):
    def issue(i):
      row = idx_ref[0, 0, i]
      pltpu.make_async_copy(table_ref.at[pl.ds(row, 1)],
                            o_ref.at[pl.ds(i, 1)], sem).start()

    @pl.loop(0, r)
    def _(i):
      issue(i)

    @pl.loop(0, r)
    def _(i):
      pltpu.make_async_copy(table_ref.at[pl.ds(0, 1)],
                            o_ref.at[pl.ds(0, 1)], sem).wait()

  return pl.pallas_call(
      body,
      grid=(g,),
      in_specs=[
          pl.BlockSpec((1, 1, r), lambda i: (i, 0, 0),
                       memory_space=pltpu.SMEM),
          pl.BlockSpec(memory_space=pl.ANY),
      ],
      out_specs=pl.BlockSpec((r, d), lambda i: (i, 0)),
      out_shape=jax.ShapeDtypeStruct((b_total, d), jnp.float32),
      scratch_shapes=[pltpu.SemaphoreType.DMA],
  )(idx3, table)


def kernel(x, table):
  b, t = x.shape
  vocab = table.shape[0]
  idx = x.reshape(-1).astype(jnp.int32)
  b_total = idx.shape[0]
  n_sc = b_total - _TC_ROWS
  idx_sc = idx[:n_sc].reshape(_NW, n_sc // _NW // _CH, _CH)
  idx_tc = idx[n_sc:].reshape(_TC_ROWS // _R, 1, _R)
  o_sc = _sc_gather(table, idx_sc)
  o_tc = _tc_gather(table, idx_tc)
  out = jnp.concatenate([o_sc, o_tc], axis=0)
  return out.reshape(b, t, vocab)


# speedup vs baseline: 2.1932x; 1.9138x over previous
import functools

import jax
import jax.numpy as jnp
from jax import lax
from jax.experimental import pallas as pl
from jax.experimental.pallas import tpu as pltpu
import jax.experimental.pallas.tpu_sc as plsc

_NC = 2
_NS = 16
_NW = _NC * _NS

_CH = 4
_NBUF = 3


def _sc_gather(table, idx3):
  nw, nch, ch = idx3.shape
  d = table.shape[1]
  b_total = nw * nch * ch
  mesh = plsc.VectorSubcoreMesh(core_axis_name="c", subcore_axis_name="s")

  @functools.partial(
      pl.kernel,
      out_type=jax.ShapeDtypeStruct((b_total, d), jnp.float32),
      mesh=mesh,
      scratch_types=[
          pltpu.VMEM((nch, ch), jnp.int32),
          *[pltpu.VMEM((ch, d), jnp.float32) for _ in range(_NBUF)],
          *[pltpu.SemaphoreType.DMA for _ in range(2 * _NBUF)],
      ],
  )
  def k(table_hbm, idx_hbm, out_hbm, idx_v, *rest):
    bufs = rest[:_NBUF]
    gsems = rest[_NBUF:2 * _NBUF]
    ssems = rest[2 * _NBUF:]
    wid = lax.axis_index("s") * _NC + lax.axis_index("c")
    base_row = wid * (nch * ch)

    pltpu.sync_copy(idx_hbm.at[wid], idx_v)

    def gather_start(b, g):
      pltpu.async_copy(table_hbm.at[idx_v.at[g]], bufs[b], gsems[b])

    def gather_wait(b):
      pltpu.make_async_copy(table_hbm.at[idx_v.at[0]], bufs[b],
                            gsems[b]).wait()

    def scatter_start(b, g):
      pltpu.async_copy(bufs[b], out_hbm.at[pl.ds(base_row + g * ch, ch)],
                       ssems[b])

    def scatter_wait(b):
      pltpu.make_async_copy(bufs[b], out_hbm.at[pl.ds(0, ch)],
                            ssems[b]).wait()

    def drain(b, g):
      gather_wait(b)
      scatter_start(b, g)
      scatter_wait(b)

    for b in range(_NBUF):
      gather_start(b, b)

    n_main = nch // _NBUF
    n_tail = nch - n_main * _NBUF

    @pl.loop(0, n_main)
    def _(o):
      for b in range(_NBUF):
        g = o * _NBUF + b
        drain(b, g)
        nxt = g + _NBUF

        @pl.when(nxt < nch)
        def _():
          gather_start(b, nxt)

    for b in range(n_tail):
      drain(b, n_main * _NBUF + b)

  return k(table, idx3)


def kernel(x, table):
  b, t = x.shape
  vocab = table.shape[0]
  idx = x.reshape(-1).astype(jnp.int32)
  b_total = idx.shape[0]
  r = b_total // _NW
  idx3 = idx.reshape(_NW, r // _CH, _CH)
  out = _sc_gather(table, idx3)
  return out.reshape(b, t, vocab)
